# baseline (speedup 1.0000x reference)
import functools

import jax
import jax.numpy as jnp
from jax import lax
from jax.experimental import pallas as pl
from jax.experimental.pallas import tpu as pltpu
from jax.experimental.pallas import tpu_sc as plsc

VOCAB_SIZE = 50257
EMBED = 768
BATCH = 4
SEQ = 2048
NTOK = BATCH * SEQ

NUM_CORES = 2
NUM_SUBCORES = 16
NUM_WORKERS = NUM_CORES * NUM_SUBCORES
LANES = 16

POS_BLK = SEQ // NUM_WORKERS
ROWS = 32
NBLK = (POS_BLK // ROWS) * BATCH
COLS = EMBED // LANES

_mesh = plsc.VectorSubcoreMesh(core_axis_name="c", subcore_axis_name="s")


@functools.partial(
    pl.kernel,
    mesh=_mesh,
    out_type=jax.ShapeDtypeStruct((NTOK, EMBED), jnp.float32),
    scratch_types=[
        pltpu.VMEM((BATCH * POS_BLK,), jnp.int32),
        pltpu.VMEM((POS_BLK, EMBED), jnp.float32),
        pltpu.VMEM((ROWS, EMBED), jnp.float32),
        pltpu.VMEM((ROWS, EMBED), jnp.float32),
        pltpu.SemaphoreType.DMA,
        pltpu.SemaphoreType.DMA,
        pltpu.SemaphoreType.DMA,
        pltpu.SemaphoreType.DMA,
        pltpu.SemaphoreType.DMA,
        pltpu.SemaphoreType.DMA,
    ],
)
def _embed_sc(x_hbm, tok_hbm, pos_hbm, out_hbm,
              idx_v, pos_v, tok0, tok1,
              isem, psem, gs0, gs1, os0, os1):
    wid = lax.axis_index("s") * NUM_CORES + lax.axis_index("c")
    pbase = wid * POS_BLK

    bufs = (tok0, tok1)
    gsems = (gs0, gs1)
    osems = (os0, os1)

    h_pos = pltpu.async_copy(pos_hbm.at[pl.ds(pbase, POS_BLK)], pos_v, psem)
    h_idx = [
        pltpu.async_copy(x_hbm.at[pl.ds(b * SEQ + pbase, POS_BLK)],
                         idx_v.at[pl.ds(b * POS_BLK, POS_BLK)], isem)
        for b in range(BATCH)
    ]

    def idx_slice(k):
        return idx_v.at[pl.ds(POS_BLK * (k // 2) + ROWS * (k % 2), ROWS)]

    def out_slice(k):
        return out_hbm.at[pl.ds((k // 2) * SEQ + pbase + ROWS * (k % 2), ROWS)]

    for h in h_idx:
        h.wait()
    g = [None] * NBLK
    o = [None] * NBLK
    g[0] = pltpu.async_copy(tok_hbm.at[idx_slice(0)], bufs[0], gsems[0])
    h_pos.wait()

    for k in range(NBLK):
        cur = k & 1
        if k + 1 < NBLK:
            if k >= 1:
                o[k - 1].wait()
            g[k + 1] = pltpu.async_copy(
                tok_hbm.at[idx_slice(k + 1)], bufs[(k + 1) & 1],
                gsems[(k + 1) & 1])
        g[k].wait()

        buf = bufs[cur]
        poff = ROWS * (k & 1)

        def _row(r, carry, buf=buf, poff=poff):
            for c in range(COLS):
                sl = pl.ds(c * LANES, LANES)
                buf[r, sl] = buf[r, sl] + pos_v[poff + r, sl]
            return carry

        lax.fori_loop(0, ROWS, _row, 0)
        o[k] = pltpu.async_copy(buf, out_slice(k), osems[cur])

    o[NBLK - 2].wait()
    o[NBLK - 1].wait()


@jax.jit
def kernel(x, tok_table, pos_table):
    out = _embed_sc(x.reshape(-1), tok_table, pos_table)
    return out.reshape(BATCH, SEQ, EMBED)


# speedup vs baseline: 1.0803x; 1.0803x over previous
import functools

import jax
import jax.numpy as jnp
from jax import lax
from jax.experimental import pallas as pl
from jax.experimental.pallas import tpu as pltpu
from jax.experimental.pallas import tpu_sc as plsc

VOCAB_SIZE = 50257
EMBED = 768
BATCH = 4
SEQ = 2048
NTOK = BATCH * SEQ

NUM_CORES = 2
NUM_SUBCORES = 16
NUM_WORKERS = NUM_CORES * NUM_SUBCORES
LANES = 16

POS_BLK = SEQ // NUM_WORKERS
ROWS = 32
NBLK = (POS_BLK // ROWS) * BATCH
COLS = EMBED // LANES

_mesh = plsc.VectorSubcoreMesh(core_axis_name="c", subcore_axis_name="s")


@functools.partial(
    pl.kernel,
    mesh=_mesh,
    out_type=jax.ShapeDtypeStruct((NTOK, EMBED), jnp.float32),
    scratch_types=[
        pltpu.VMEM((BATCH * POS_BLK,), jnp.int32),
        pltpu.VMEM((POS_BLK, EMBED), jnp.float32),
        pltpu.VMEM((ROWS, EMBED), jnp.float32),
        pltpu.VMEM((ROWS, EMBED), jnp.float32),
        pltpu.VMEM((ROWS, EMBED), jnp.float32),
        pltpu.SemaphoreType.DMA,
        pltpu.SemaphoreType.DMA,
        pltpu.SemaphoreType.DMA,
        pltpu.SemaphoreType.DMA,
        pltpu.SemaphoreType.DMA,
        pltpu.SemaphoreType.DMA,
        pltpu.SemaphoreType.DMA,
        pltpu.SemaphoreType.DMA,
    ],
)
def _embed_sc(x_hbm, tok_hbm, pos_hbm, out_hbm,
              idx_v, pos_v, tok0, tok1, tok2,
              isem, psem, gs0, gs1, gs2, os0, os1, os2):
    wid = lax.axis_index("s") * NUM_CORES + lax.axis_index("c")
    pbase = wid * POS_BLK

    bufs = (tok0, tok1, tok2)
    gsems = (gs0, gs1, gs2)
    osems = (os0, os1, os2)
    NBUF = 3

    h_pos = pltpu.async_copy(pos_hbm.at[pl.ds(pbase, POS_BLK)], pos_v, psem)
    h_idx = [
        pltpu.async_copy(x_hbm.at[pl.ds(b * SEQ + pbase, POS_BLK)],
                         idx_v.at[pl.ds(b * POS_BLK, POS_BLK)], isem)
        for b in range(BATCH)
    ]

    def idx_slice(k):
        return idx_v.at[pl.ds(POS_BLK * (k // 2) + ROWS * (k % 2), ROWS)]

    def out_slice(k):
        return out_hbm.at[pl.ds((k // 2) * SEQ + pbase + ROWS * (k % 2), ROWS)]

    for h in h_idx:
        h.wait()
    g = [None] * NBLK
    o = [None] * NBLK
    g[0] = pltpu.async_copy(tok_hbm.at[idx_slice(0)], bufs[0], gsems[0])
    g[1] = pltpu.async_copy(tok_hbm.at[idx_slice(1)], bufs[1], gsems[1])
    h_pos.wait()

    for k in range(NBLK):
        cur = k % NBUF
        if k + 2 < NBLK:
            nxt = (k + 2) % NBUF
            if k >= 1:
                o[k - 1].wait()
            g[k + 2] = pltpu.async_copy(
                tok_hbm.at[idx_slice(k + 2)], bufs[nxt], gsems[nxt])
        g[k].wait()

        buf = bufs[cur]
        poff = ROWS * (k & 1)

        def _row(r, carry, buf=buf, poff=poff):
            for c in range(COLS):
                sl = pl.ds(c * LANES, LANES)
                buf[r, sl] = buf[r, sl] + pos_v[poff + r, sl]
            return carry

        lax.fori_loop(0, ROWS, _row, 0)
        o[k] = pltpu.async_copy(buf, out_slice(k), osems[cur])

    o[NBLK - 3].wait()
    o[NBLK - 2].wait()
    o[NBLK - 1].wait()


@jax.jit
def kernel(x, tok_table, pos_table):
    out = _embed_sc(x.reshape(-1), tok_table, pos_table)
    return out.reshape(BATCH, SEQ, EMBED)
